# baseline (speedup 1.0000x reference)
import functools

import jax
import jax.numpy as jnp
from jax import lax
from jax.experimental import pallas as pl
from jax.experimental.pallas import tpu as pltpu
from jax.experimental.pallas import tpu_sc as plsc

_NUM_WORKERS = 32
_LANES = 16


@functools.lru_cache(maxsize=None)
def _make_gather_kernel(B, V, D, chunk=16, nbuf=8, look=4):
  bpw = B // _NUM_WORKERS
  nchunk = bpw // chunk
  mesh = plsc.VectorSubcoreMesh(core_axis_name="c", subcore_axis_name="s")

  @functools.partial(
      pl.kernel,
      mesh=mesh,
      out_type=jax.ShapeDtypeStruct((B, D), jnp.float32),
      scratch_types=[
          pltpu.VMEM((bpw,), jnp.int32),
          pltpu.VMEM((bpw,), jnp.int32),
          pltpu.VMEM((bpw,), jnp.int32),
          [pltpu.VMEM((chunk, D), jnp.float32)] * nbuf,
          [pltpu.SemaphoreType.DMA] * nbuf,
          [pltpu.SemaphoreType.DMA] * nbuf,
      ],
  )
  def gather_kernel(i1_hbm, i2_hbm, table_hbm, out_hbm,
                    i1_v, i2_v, flat_v, bufs, gsems, psems):
    wid = lax.axis_index("s") * 2 + lax.axis_index("c")
    base = wid * bpw

    c1 = pltpu.async_copy(i1_hbm.at[pl.ds(base, bpw)], i1_v, gsems[0])
    c2 = pltpu.async_copy(i2_hbm.at[pl.ds(base, bpw)], i2_v, gsems[1])
    c1.wait()
    c2.wait()

    def compute_flat(t):
      for j in range(t * chunk // _LANES, (t + 1) * chunk // _LANES):
        sl = pl.ds(j * _LANES, _LANES)
        flat_v[sl] = i1_v[sl] * V + i2_v[sl]

    gcopy = [None] * nbuf
    pcopy = [None] * nbuf
    put_waited = [True] * nbuf
    for t in range(nchunk + look):
      if t < nchunk:
        b = t % nbuf
        if not put_waited[b]:
          pcopy[b].wait()
          put_waited[b] = True
        compute_flat(t)
        gcopy[b] = pltpu.async_copy(
            table_hbm.at[flat_v.at[pl.ds(t * chunk, chunk)]],
            bufs[b], gsems[b])
      c = t - look
      if c >= 0:
        b = c % nbuf
        gcopy[b].wait()
        pcopy[b] = pltpu.async_copy(
            bufs[b], out_hbm.at[pl.ds(base + c * chunk, chunk)], psems[b])
        put_waited[b] = False
    for b in range(nbuf):
      if not put_waited[b]:
        pcopy[b].wait()

  return gather_kernel


def kernel(idx, table):
  B = idx.shape[0]
  VV, D = table.shape
  V = int(round(VV ** 0.5))
  idx32 = idx.astype(jnp.int32)
  return _make_gather_kernel(B, V, D)(idx32[:, 0], idx32[:, 1], table)


# speedup vs baseline: 1.0122x; 1.0122x over previous
import functools

import jax
import jax.numpy as jnp
from jax import lax
from jax.experimental import pallas as pl
from jax.experimental.pallas import tpu as pltpu
from jax.experimental.pallas import tpu_sc as plsc

_NUM_WORKERS = 32
_LANES = 16


@functools.lru_cache(maxsize=None)
def _make_gather_kernel(B, V, D, chunk=32, nbuf=6, look=3):
  bpw = B // _NUM_WORKERS
  nchunk = bpw // chunk
  mesh = plsc.VectorSubcoreMesh(core_axis_name="c", subcore_axis_name="s")

  @functools.partial(
      pl.kernel,
      mesh=mesh,
      out_type=jax.ShapeDtypeStruct((B, D), jnp.float32),
      scratch_types=[
          pltpu.VMEM((bpw,), jnp.int32),
          pltpu.VMEM((bpw,), jnp.int32),
          pltpu.VMEM((bpw,), jnp.int32),
          [pltpu.VMEM((chunk, D), jnp.float32)] * nbuf,
          [pltpu.SemaphoreType.DMA] * nbuf,
          [pltpu.SemaphoreType.DMA] * nbuf,
      ],
  )
  def gather_kernel(i1_hbm, i2_hbm, table_hbm, out_hbm,
                    i1_v, i2_v, flat_v, bufs, gsems, psems):
    wid = lax.axis_index("s") * 2 + lax.axis_index("c")
    base = wid * bpw

    c1 = pltpu.async_copy(i1_hbm.at[pl.ds(base, bpw)], i1_v, gsems[0])
    c2 = pltpu.async_copy(i2_hbm.at[pl.ds(base, bpw)], i2_v, gsems[1])
    c1.wait()
    c2.wait()

    def compute_flat(t):
      for j in range(t * chunk // _LANES, (t + 1) * chunk // _LANES):
        sl = pl.ds(j * _LANES, _LANES)
        flat_v[sl] = i1_v[sl] * V + i2_v[sl]

    gcopy = [None] * nbuf
    pcopy = [None] * nbuf
    put_waited = [True] * nbuf
    for t in range(nchunk + look):
      if t < nchunk:
        b = t % nbuf
        if not put_waited[b]:
          pcopy[b].wait()
          put_waited[b] = True
        compute_flat(t)
        gcopy[b] = pltpu.async_copy(
            table_hbm.at[flat_v.at[pl.ds(t * chunk, chunk)]],
            bufs[b], gsems[b])
      c = t - look
      if c >= 0:
        b = c % nbuf
        gcopy[b].wait()
        pcopy[b] = pltpu.async_copy(
            bufs[b], out_hbm.at[pl.ds(base + c * chunk, chunk)], psems[b])
        put_waited[b] = False
    for b in range(nbuf):
      if not put_waited[b]:
        pcopy[b].wait()

  return gather_kernel


def kernel(idx, table):
  B = idx.shape[0]
  VV, D = table.shape
  V = int(round(VV ** 0.5))
  idx32 = idx.astype(jnp.int32)
  return _make_gather_kernel(B, V, D)(idx32[:, 0], idx32[:, 1], table)


# speedup vs baseline: 1.0209x; 1.0086x over previous
import functools

import jax
import jax.numpy as jnp
from jax import lax
from jax.experimental import pallas as pl
from jax.experimental.pallas import tpu as pltpu
from jax.experimental.pallas import tpu_sc as plsc

_NUM_WORKERS = 32
_LANES = 16


@functools.lru_cache(maxsize=None)
def _make_gather_kernel(B, V, D, chunk=64, nbuf=3, look=2):
  bpw = B // _NUM_WORKERS
  nchunk = bpw // chunk
  mesh = plsc.VectorSubcoreMesh(core_axis_name="c", subcore_axis_name="s")

  @functools.partial(
      pl.kernel,
      mesh=mesh,
      out_type=jax.ShapeDtypeStruct((B, D), jnp.float32),
      scratch_types=[
          pltpu.VMEM((bpw,), jnp.int32),
          pltpu.VMEM((bpw,), jnp.int32),
          pltpu.VMEM((bpw,), jnp.int32),
          [pltpu.VMEM((chunk, D), jnp.float32)] * nbuf,
          [pltpu.SemaphoreType.DMA] * nbuf,
          [pltpu.SemaphoreType.DMA] * nbuf,
      ],
  )
  def gather_kernel(i1_hbm, i2_hbm, table_hbm, out_hbm,
                    i1_v, i2_v, flat_v, bufs, gsems, psems):
    wid = lax.axis_index("s") * 2 + lax.axis_index("c")
    base = wid * bpw

    c1 = pltpu.async_copy(i1_hbm.at[pl.ds(base, bpw)], i1_v, gsems[0])
    c2 = pltpu.async_copy(i2_hbm.at[pl.ds(base, bpw)], i2_v, gsems[1])
    c1.wait()
    c2.wait()

    def compute_flat(t):
      for j in range(t * chunk // _LANES, (t + 1) * chunk // _LANES):
        sl = pl.ds(j * _LANES, _LANES)
        flat_v[sl] = i1_v[sl] * V + i2_v[sl]

    gcopy = [None] * nbuf
    pcopy = [None] * nbuf
    put_waited = [True] * nbuf
    for t in range(nchunk + look):
      if t < nchunk:
        b = t % nbuf
        if not put_waited[b]:
          pcopy[b].wait()
          put_waited[b] = True
        compute_flat(t)
        gcopy[b] = pltpu.async_copy(
            table_hbm.at[flat_v.at[pl.ds(t * chunk, chunk)]],
            bufs[b], gsems[b])
      c = t - look
      if c >= 0:
        b = c % nbuf
        gcopy[b].wait()
        pcopy[b] = pltpu.async_copy(
            bufs[b], out_hbm.at[pl.ds(base + c * chunk, chunk)], psems[b])
        put_waited[b] = False
    for b in range(nbuf):
      if not put_waited[b]:
        pcopy[b].wait()

  return gather_kernel


def kernel(idx, table):
  B = idx.shape[0]
  VV, D = table.shape
  V = int(round(VV ** 0.5))
  idx32 = idx.astype(jnp.int32)
  return _make_gather_kernel(B, V, D)(idx32[:, 0], idx32[:, 1], table)
